# baseline (speedup 1.0000x reference)
import functools

import jax
import jax.numpy as jnp
from jax import lax
from jax.experimental import pallas as pl
from jax.experimental.pallas import tpu as pltpu
from jax.experimental.pallas import tpu_sc as plsc

B = 2
N = 10000
D = 128
E = 320000
H = 4
OUT = 32
NP = 10240
NTP = B * NP
AW = 128
DW = 16

NC = 2
NS = 16
K = 32
EPT = 20480
EP = NS * EPT
NCHUNK = EPT // K
DR = NP // 8
ACCR = NP + DR
RPT = ACCR // NS
ZROWS = 4
MS = NP // NS
DS = DR // NS

_f32 = jnp.float32


def _proj_body(x_ref, w_ref, as_ref, ad_ref, h_ref, asp_ref, adp_ref):
    h = jnp.dot(x_ref[...], w_ref[...], preferred_element_type=_f32)
    h_ref[...] = h
    asp_ref[...] = jnp.dot(h, as_ref[...], preferred_element_type=_f32)
    adp_ref[...] = jnp.dot(h, ad_ref[...], preferred_element_type=_f32)


def _epilogue(num, den16, rden, b, g, bt):
    den = jnp.dot(den16, rden, preferred_element_type=_f32)
    x = num / (den + 1e-16) + b
    mu = jnp.mean(x, axis=-1, keepdims=True)
    var = jnp.mean((x - mu) ** 2, axis=-1, keepdims=True)
    x = (x - mu) / jnp.sqrt(var + 1e-5) * g + bt
    return jnp.maximum(x, 0.0)


def _mid_body(msg_ref, den_ref, rden_ref, b_ref, g_ref, bt_ref, w_ref, as_ref,
              ad_ref, h_ref, asp_ref, adp_ref):
    x = _epilogue(msg_ref[...], den_ref[...], rden_ref[...], b_ref[...],
                  g_ref[...], bt_ref[...])
    h = jnp.dot(x, w_ref[...], preferred_element_type=_f32)
    h_ref[...] = h
    asp_ref[...] = jnp.dot(h, as_ref[...], preferred_element_type=_f32)
    adp_ref[...] = jnp.dot(h, ad_ref[...], preferred_element_type=_f32)


def _last_body(msg_ref, den_ref, rden_ref, b_ref, g_ref, bt_ref, o_ref):
    o_ref[...] = _epilogue(msg_ref[...], den_ref[...], rden_ref[...], b_ref[...],
                           g_ref[...], bt_ref[...])


_M = 2048


def _full(shape):
    return pl.BlockSpec(shape, lambda i: (0, 0))


def _rows(w):
    return pl.BlockSpec((_M, w), lambda i: (i, 0))


def _proj(x, w, a_s, a_d):
    return pl.pallas_call(
        _proj_body,
        grid=(NTP // _M,),
        in_specs=[_rows(128), _full((128, 128)), _full((128, AW)),
                  _full((128, AW))],
        out_specs=[_rows(128), _rows(AW), _rows(AW)],
        out_shape=[jax.ShapeDtypeStruct((NTP, 128), _f32),
                   jax.ShapeDtypeStruct((NTP, AW), _f32),
                   jax.ShapeDtypeStruct((NTP, AW), _f32)],
    )(x, w, a_s, a_d)


def _mid(msg, den16, rden, b, g, bt, w, a_s, a_d):
    return pl.pallas_call(
        _mid_body,
        grid=(NTP // _M,),
        in_specs=[_rows(128), _rows(DW), _full((DW, 128)), _full((1, 128)),
                  _full((1, 128)), _full((1, 128)), _full((128, 128)),
                  _full((128, AW)), _full((128, AW))],
        out_specs=[_rows(128), _rows(AW), _rows(AW)],
        out_shape=[jax.ShapeDtypeStruct((NTP, 128), _f32),
                   jax.ShapeDtypeStruct((NTP, AW), _f32),
                   jax.ShapeDtypeStruct((NTP, AW), _f32)],
    )(msg, den16, rden, b, g, bt, w, a_s, a_d)


def _last(msg, den16, rden, b, g, bt):
    return pl.pallas_call(
        _last_body,
        grid=(NTP // _M,),
        in_specs=[_rows(128), _rows(DW), _full((DW, 128)), _full((1, 128)),
                  _full((1, 128)), _full((1, 128))],
        out_specs=_rows(128),
        out_shape=jax.ShapeDtypeStruct((NTP, 128), _f32),
    )(msg, den16, rden, b, g, bt)


def _edge_body(h_hbm, asp_hbm, adp_hbm, src_hbm, dst_hbm, msg_hbm, den_hbm,
               srcv0, srcv1, dstv0, dstv1, dgv0, dgv1, didx, mdi0, mdi1,
               exb, hbuf0, hbuf1, aspbuf0, aspbuf1, adpbuf0, adpbuf1,
               sbuf0, sbuf1, sbuf2, acc,
               sgi0, sgi1, sgd0, sgd1, msc0, msc1, dsc):
    c = lax.axis_index("c")
    s = lax.axis_index("s")
    zeros16 = jnp.zeros((16,), _f32)
    srcv = (srcv0, srcv1)
    dstv = (dstv0, dstv1)
    dgv = (dgv0, dgv1)
    mdi = (mdi0, mdi1)
    hbuf = (hbuf0, hbuf1)
    aspbuf = (aspbuf0, aspbuf1)
    adpbuf = (adpbuf0, adpbuf1)
    sbuf = (sbuf0, sbuf1)
    sgi = (sgi0, sgi1)
    sgd = (sgd0, sgd1)
    msc = (msc0, msc1)

    def z2(i, carry):
        for j in range(8):
            sbuf2[i, 16 * j:16 * (j + 1)] = zeros16
        return carry

    lax.fori_loop(0, K, z2, 0)
    for k in range(RPT // K):
        pltpu.sync_copy(sbuf2, acc.at[pl.ds(s * RPT + k * K, K)])
    pltpu.sync_copy(sbuf2.at[pl.ds(0, RPT % K)],
                    acc.at[pl.ds(s * RPT + (RPT // K) * K, RPT % K)])
    plsc.subcore_barrier()

    splats = [jnp.full((16,), j, jnp.int32) for j in range(H)]
    base = s * EPT

    def idx_load(n, p):
        off = base + n * K
        pltpu.async_copy(src_hbm.at[pl.ds(c * EP + off, K)], srcv[p], sgi[p])
        pltpu.async_copy(dst_hbm.at[pl.ds(off, K)], dstv[p], sgi[p])

    def idx_wait(n, p):
        off = base + n * K
        pltpu.make_async_copy(src_hbm.at[pl.ds(c * EP + off, K)], srcv[p],
                              sgi[p]).wait()
        pltpu.make_async_copy(dst_hbm.at[pl.ds(off, K)], dstv[p],
                              sgi[p]).wait()

    def issue(p):
        for q in range(K // 16):
            dv = dstv[p][pl.ds(16 * q, 16)]
            dgv[p][16 * q:16 * (q + 1)] = dv + c * jnp.int32(NP)
        pltpu.async_copy(h_hbm.at[srcv[p]], hbuf[p], sgd[p])
        pltpu.async_copy(asp_hbm.at[srcv[p]], aspbuf[p], sgd[p])
        pltpu.async_copy(adp_hbm.at[dgv[p]], adpbuf[p], sgd[p])

    def gather_wait(p):
        pltpu.make_async_copy(h_hbm.at[srcv[p]], hbuf[p], sgd[p]).wait()
        pltpu.make_async_copy(asp_hbm.at[srcv[p]], aspbuf[p], sgd[p]).wait()
        pltpu.make_async_copy(adp_hbm.at[dgv[p]], adpbuf[p], sgd[p]).wait()

    def drain_msg(p):
        pltpu.make_async_copy(sbuf[p], acc.at[mdi[p]], msc[p]).wait()

    def drain_den_and_clear(p_prev):
        pltpu.make_async_copy(sbuf2, acc.at[didx], dsc).wait()

        def clr(q):
            lv = (mdi[p_prev][pl.ds(q, 16)] & 7) * 16
            for i in range(16):
                sbuf2[q + i, pl.ds(lv[i], 16)] = zeros16

        plsc.parallel_loop(0, K, 16, unroll=2)(clr)

    def compute(p, first_den, last):
        hb, ab, db = hbuf[p], aspbuf[p], adpbuf[p]
        for q in range(K // 16):
            dv = dstv[p][pl.ds(16 * q, 16)]
            didx[16 * q:16 * (q + 1)] = jnp.int32(NP) + (dv >> 3)
            mdi[p][16 * q:16 * (q + 1)] = dv

        def expo(q):
            for i in range(16):
                e = q + i
                a = ab[e, 0:16] + db[e, 0:16]
                a = jnp.maximum(a, 0.2 * a)
                exb[pl.ds(q * 16 + i * 16, 16)] = jnp.exp(a)

        plsc.parallel_loop(0, K, 16, unroll=2)(expo)

        def scale(q):
            for i in range(16):
                e = q + i
                ex = exb[pl.ds(q * 16 + i * 16, 16)]
                for j in range(8):
                    w = ex.at[splats[j // 2]].get(mode="promise_in_bounds")
                    sbuf[p][e, 16 * j:16 * (j + 1)] = (
                        hb[e, 16 * j:16 * (j + 1)] * w)

        plsc.parallel_loop(0, K, 16, unroll=2)(scale)
        pltpu.async_copy(sbuf[p], acc.at[mdi[p]], msc[p], add=True)
        if not first_den:
            drain_den_and_clear(1 - p)

        def place(q):
            lv = (dstv[p][pl.ds(q, 16)] & 7) * 16
            for i in range(16):
                sbuf2[q + i, pl.ds(lv[i], 16)] = exb[pl.ds(q * 16 + i * 16, 16)]

        plsc.parallel_loop(0, K, 16, unroll=2)(place)
        pltpu.async_copy(sbuf2, acc.at[didx], dsc, add=True)

    idx_load(0, 0)
    idx_wait(0, 0)
    issue(0)
    idx_load(1, 1)

    def section(n, p, skip_msg=False, skip_den=False):
        op = 1 - p
        idx_wait(n + 1, op)
        issue(op)
        gather_wait(p)
        if not skip_msg:
            drain_msg(p)
        compute(p, skip_den, False)
        idx_load(n + 2, p)

    section(0, 0, skip_msg=True, skip_den=True)
    section(1, 1, skip_msg=True, skip_den=False)

    def pair(g, carry):
        n = 2 * g + 2
        section(n, 0)
        section(n + 1, 1)
        return carry

    lax.fori_loop(0, (NCHUNK - 4) // 2, pair, 0)
    idx_wait(NCHUNK - 1, 1)
    issue(1)
    gather_wait(0)
    drain_msg(0)
    compute(0, False, False)
    gather_wait(1)
    drain_msg(1)
    compute(1, False, True)
    drain_msg(0)
    drain_msg(1)
    pltpu.make_async_copy(sbuf2, acc.at[didx], dsc).wait()

    plsc.subcore_barrier()
    pltpu.sync_copy(acc.at[pl.ds(s * MS, MS)],
                    msg_hbm.at[pl.ds(c * NP + s * MS, MS)])
    pltpu.sync_copy(acc.at[pl.ds(NP + s * DS, DS)],
                    den_hbm.at[pl.ds(c * DR + s * DS, DS)])


_edge_call = functools.partial(
    pl.kernel,
    mesh=plsc.VectorSubcoreMesh(core_axis_name="c", subcore_axis_name="s"),
    out_type=[jax.ShapeDtypeStruct((NTP, 128), _f32),
              jax.ShapeDtypeStruct((B * DR, 128), _f32)],
    scratch_types=[
        pltpu.VMEM((K,), jnp.int32),
        pltpu.VMEM((K,), jnp.int32),
        pltpu.VMEM((K,), jnp.int32),
        pltpu.VMEM((K,), jnp.int32),
        pltpu.VMEM((K,), jnp.int32),
        pltpu.VMEM((K,), jnp.int32),
        pltpu.VMEM((K,), jnp.int32),
        pltpu.VMEM((K,), jnp.int32),
        pltpu.VMEM((K,), jnp.int32),
        pltpu.VMEM((K * 16,), _f32),
        pltpu.VMEM((K, 128), _f32),
        pltpu.VMEM((K, 128), _f32),
        pltpu.VMEM((K, AW), _f32),
        pltpu.VMEM((K, AW), _f32),
        pltpu.VMEM((K, AW), _f32),
        pltpu.VMEM((K, AW), _f32),
        pltpu.VMEM((K, 128), _f32),
        pltpu.VMEM((K, 128), _f32),
        pltpu.VMEM((K, 128), _f32),
        pltpu.VMEM_SHARED((ACCR, 128), _f32),
        pltpu.SemaphoreType.DMA,
        pltpu.SemaphoreType.DMA,
        pltpu.SemaphoreType.DMA,
        pltpu.SemaphoreType.DMA,
        pltpu.SemaphoreType.DMA,
        pltpu.SemaphoreType.DMA,
        pltpu.SemaphoreType.DMA,
    ],
)(_edge_body)


def _att_mat(a):
    m = jnp.zeros((D, AW), _f32)
    for hh in range(H):
        m = m.at[hh * OUT:(hh + 1) * OUT, hh].set(a[hh])
    return m


def kernel(feature, edge_index, W1, att_src1, att_dst1, b1, g1, bt1,
           W2, att_src2, att_dst2, b2, g2, bt2):
    x = feature.reshape(B * N, D)
    x = jnp.concatenate([x[:N], jnp.zeros((NP - N, D), _f32),
                         x[N:], jnp.zeros((NP - N, D), _f32)], axis=0)
    ei = edge_index.astype(jnp.int32)
    padz = jnp.zeros((EP - E,), jnp.int32)
    padd = jnp.full((EP - E,), N + 1, jnp.int32)
    src2 = jnp.concatenate([ei[0], padz, ei[0] + jnp.int32(NP), padz])
    dst = jnp.concatenate([ei[1], padd])

    rden = jnp.concatenate(
        [jnp.kron(jnp.eye(H, dtype=_f32), jnp.ones((1, OUT), _f32)),
         jnp.zeros((DW - H, D), _f32)], axis=0)

    h1, asp1, adp1 = _proj(x, W1, _att_mat(att_src1), _att_mat(att_dst1))
    msg1, dpack1 = _edge_call(h1, asp1, adp1, src2, dst)
    h2, asp2, adp2 = _mid(msg1, dpack1.reshape(NTP, DW), rden,
                          b1.reshape(1, D), g1.reshape(1, D),
                          bt1.reshape(1, D), W2, _att_mat(att_src2),
                          _att_mat(att_dst2))
    msg2, dpack2 = _edge_call(h2, asp2, adp2, src2, dst)
    out = _last(msg2, dpack2.reshape(NTP, DW), rden, b2.reshape(1, D),
                g2.reshape(1, D), bt2.reshape(1, D))
    return out.reshape(B, NP, H * OUT)[:, :N, :]


# speedup vs baseline: 1.0071x; 1.0071x over previous
import functools

import jax
import jax.numpy as jnp
from jax import lax
from jax.experimental import pallas as pl
from jax.experimental.pallas import tpu as pltpu
from jax.experimental.pallas import tpu_sc as plsc

B = 2
N = 10000
D = 128
E = 320000
H = 4
OUT = 32
NP = 10240
NTP = B * NP
AW = 128
DW = 16

NC = 2
NS = 16
K = 32
EPT = 20480
EP = NS * EPT
NCHUNK = EPT // K
DR = NP // 8
ACCR = NP + DR
RPT = ACCR // NS
ZROWS = 4
MS = NP // NS
DS = DR // NS

_f32 = jnp.float32


def _proj_body(x_ref, w_ref, as_ref, ad_ref, hx_ref, adp_ref):
    h = jnp.dot(x_ref[...], w_ref[...], preferred_element_type=_f32)
    asp = jnp.dot(h, as_ref[...], preferred_element_type=_f32)
    hx_ref[...] = jnp.concatenate([h, asp], axis=1)
    adp_ref[...] = jnp.dot(h, ad_ref[...], preferred_element_type=_f32)


def _epilogue(num, den16, rden, b, g, bt):
    den = jnp.dot(den16, rden, preferred_element_type=_f32)
    x = num / (den + 1e-16) + b
    mu = jnp.mean(x, axis=-1, keepdims=True)
    var = jnp.mean((x - mu) ** 2, axis=-1, keepdims=True)
    x = (x - mu) / jnp.sqrt(var + 1e-5) * g + bt
    return jnp.maximum(x, 0.0)


def _mid_body(msg_ref, den_ref, rden_ref, b_ref, g_ref, bt_ref, w_ref, as_ref,
              ad_ref, hx_ref, adp_ref):
    x = _epilogue(msg_ref[...], den_ref[...], rden_ref[...], b_ref[...],
                  g_ref[...], bt_ref[...])
    h = jnp.dot(x, w_ref[...], preferred_element_type=_f32)
    asp = jnp.dot(h, as_ref[...], preferred_element_type=_f32)
    hx_ref[...] = jnp.concatenate([h, asp], axis=1)
    adp_ref[...] = jnp.dot(h, ad_ref[...], preferred_element_type=_f32)


def _last_body(msg_ref, den_ref, rden_ref, b_ref, g_ref, bt_ref, o_ref):
    o_ref[...] = _epilogue(msg_ref[...], den_ref[...], rden_ref[...], b_ref[...],
                           g_ref[...], bt_ref[...])


_M = 2048


def _full(shape):
    return pl.BlockSpec(shape, lambda i: (0, 0))


def _rows(w):
    return pl.BlockSpec((_M, w), lambda i: (i, 0))


def _proj(x, w, a_s, a_d):
    return pl.pallas_call(
        _proj_body,
        grid=(NTP // _M,),
        in_specs=[_rows(128), _full((128, 128)), _full((128, AW)),
                  _full((128, AW))],
        out_specs=[_rows(256), _rows(AW)],
        out_shape=[jax.ShapeDtypeStruct((NTP, 256), _f32),
                   jax.ShapeDtypeStruct((NTP, AW), _f32)],
    )(x, w, a_s, a_d)


def _mid(msg, den16, rden, b, g, bt, w, a_s, a_d):
    return pl.pallas_call(
        _mid_body,
        grid=(NTP // _M,),
        in_specs=[_rows(128), _rows(DW), _full((DW, 128)), _full((1, 128)),
                  _full((1, 128)), _full((1, 128)), _full((128, 128)),
                  _full((128, AW)), _full((128, AW))],
        out_specs=[_rows(256), _rows(AW)],
        out_shape=[jax.ShapeDtypeStruct((NTP, 256), _f32),
                   jax.ShapeDtypeStruct((NTP, AW), _f32)],
    )(msg, den16, rden, b, g, bt, w, a_s, a_d)


def _last(msg, den16, rden, b, g, bt):
    return pl.pallas_call(
        _last_body,
        grid=(NTP // _M,),
        in_specs=[_rows(128), _rows(DW), _full((DW, 128)), _full((1, 128)),
                  _full((1, 128)), _full((1, 128))],
        out_specs=_rows(128),
        out_shape=jax.ShapeDtypeStruct((NTP, 128), _f32),
    )(msg, den16, rden, b, g, bt)


def _edge_body(hx_hbm, adp_hbm, src_hbm, dst_hbm, msg_hbm, den_hbm,
               srcv0, srcv1, dstv0, dstv1, dgv0, dgv1, didx, mdi0, mdi1,
               exb, hbuf0, hbuf1, adpbuf0, adpbuf1,
               sbuf0, sbuf1, sbuf2, acc,
               sgi0, sgi1, sgd0, sgd1, msc0, msc1, dsc):
    c = lax.axis_index("c")
    s = lax.axis_index("s")
    zeros16 = jnp.zeros((16,), _f32)
    srcv = (srcv0, srcv1)
    dstv = (dstv0, dstv1)
    dgv = (dgv0, dgv1)
    mdi = (mdi0, mdi1)
    hbuf = (hbuf0, hbuf1)
    adpbuf = (adpbuf0, adpbuf1)
    sbuf = (sbuf0, sbuf1)
    sgi = (sgi0, sgi1)
    sgd = (sgd0, sgd1)
    msc = (msc0, msc1)

    def z2(i, carry):
        for j in range(8):
            sbuf2[i, 16 * j:16 * (j + 1)] = zeros16
        return carry

    lax.fori_loop(0, K, z2, 0)
    for k in range(RPT // K):
        pltpu.sync_copy(sbuf2, acc.at[pl.ds(s * RPT + k * K, K)])
    pltpu.sync_copy(sbuf2.at[pl.ds(0, RPT % K)],
                    acc.at[pl.ds(s * RPT + (RPT // K) * K, RPT % K)])
    plsc.subcore_barrier()

    splats = [jnp.full((16,), j, jnp.int32) for j in range(H)]
    base = s * EPT

    def idx_load(n, p):
        off = base + n * K
        pltpu.async_copy(src_hbm.at[pl.ds(c * EP + off, K)], srcv[p], sgi[p])
        pltpu.async_copy(dst_hbm.at[pl.ds(off, K)], dstv[p], sgi[p])

    def idx_wait(n, p):
        off = base + n * K
        pltpu.make_async_copy(src_hbm.at[pl.ds(c * EP + off, K)], srcv[p],
                              sgi[p]).wait()
        pltpu.make_async_copy(dst_hbm.at[pl.ds(off, K)], dstv[p],
                              sgi[p]).wait()

    def issue(p):
        for q in range(K // 16):
            dv = dstv[p][pl.ds(16 * q, 16)]
            dgv[p][16 * q:16 * (q + 1)] = dv + c * jnp.int32(NP)
        pltpu.async_copy(hx_hbm.at[srcv[p]], hbuf[p], sgd[p])
        pltpu.async_copy(adp_hbm.at[dgv[p]], adpbuf[p], sgd[p])

    def gather_wait(p):
        pltpu.make_async_copy(hx_hbm.at[srcv[p]], hbuf[p], sgd[p]).wait()
        pltpu.make_async_copy(adp_hbm.at[dgv[p]], adpbuf[p], sgd[p]).wait()

    def drain_msg(p):
        pltpu.make_async_copy(sbuf[p], acc.at[mdi[p]], msc[p]).wait()

    def drain_den_and_clear(p_prev):
        pltpu.make_async_copy(sbuf2, acc.at[didx], dsc).wait()

        def clr(q):
            lv = (mdi[p_prev][pl.ds(q, 16)] & 7) * 16
            for i in range(16):
                sbuf2[q + i, pl.ds(lv[i], 16)] = zeros16

        plsc.parallel_loop(0, K, 16, unroll=2)(clr)

    def compute(p, first_den, last):
        hb, db = hbuf[p], adpbuf[p]
        for q in range(K // 16):
            dv = dstv[p][pl.ds(16 * q, 16)]
            didx[16 * q:16 * (q + 1)] = jnp.int32(NP) + (dv >> 3)
            mdi[p][16 * q:16 * (q + 1)] = dv

        def expo(q):
            for i in range(16):
                e = q + i
                a = hb[e, 128:144] + db[e, 0:16]
                a = jnp.maximum(a, 0.2 * a)
                exb[pl.ds(q * 16 + i * 16, 16)] = jnp.exp(a)

        plsc.parallel_loop(0, K, 16, unroll=2)(expo)

        def scale(q):
            for i in range(16):
                e = q + i
                ex = exb[pl.ds(q * 16 + i * 16, 16)]
                for j in range(8):
                    w = ex.at[splats[j // 2]].get(mode="promise_in_bounds")
                    sbuf[p][e, 16 * j:16 * (j + 1)] = (
                        hb[e, 16 * j:16 * (j + 1)] * w)

        plsc.parallel_loop(0, K, 16, unroll=2)(scale)
        pltpu.async_copy(sbuf[p], acc.at[mdi[p]], msc[p], add=True)
        if not first_den:
            drain_den_and_clear(1 - p)

        def place(q):
            lv = (dstv[p][pl.ds(q, 16)] & 7) * 16
            for i in range(16):
                sbuf2[q + i, pl.ds(lv[i], 16)] = exb[pl.ds(q * 16 + i * 16, 16)]

        plsc.parallel_loop(0, K, 16, unroll=2)(place)
        pltpu.async_copy(sbuf2, acc.at[didx], dsc, add=True)

    idx_load(0, 0)
    idx_wait(0, 0)
    issue(0)
    idx_load(1, 1)

    def section(n, p, skip_msg=False, skip_den=False):
        op = 1 - p
        idx_wait(n + 1, op)
        issue(op)
        gather_wait(p)
        if not skip_msg:
            drain_msg(p)
        compute(p, skip_den, False)
        idx_load(n + 2, p)

    section(0, 0, skip_msg=True, skip_den=True)
    section(1, 1, skip_msg=True, skip_den=False)

    def pair(g, carry):
        n = 2 * g + 2
        section(n, 0)
        section(n + 1, 1)
        return carry

    lax.fori_loop(0, (NCHUNK - 4) // 2, pair, 0)
    idx_wait(NCHUNK - 1, 1)
    issue(1)
    gather_wait(0)
    drain_msg(0)
    compute(0, False, False)
    gather_wait(1)
    drain_msg(1)
    compute(1, False, True)
    drain_msg(0)
    drain_msg(1)
    pltpu.make_async_copy(sbuf2, acc.at[didx], dsc).wait()

    plsc.subcore_barrier()
    pltpu.sync_copy(acc.at[pl.ds(s * MS, MS)],
                    msg_hbm.at[pl.ds(c * NP + s * MS, MS)])
    pltpu.sync_copy(acc.at[pl.ds(NP + s * DS, DS)],
                    den_hbm.at[pl.ds(c * DR + s * DS, DS)])


_edge_call = functools.partial(
    pl.kernel,
    mesh=plsc.VectorSubcoreMesh(core_axis_name="c", subcore_axis_name="s"),
    out_type=[jax.ShapeDtypeStruct((NTP, 128), _f32),
              jax.ShapeDtypeStruct((B * DR, 128), _f32)],
    scratch_types=[
        pltpu.VMEM((K,), jnp.int32),
        pltpu.VMEM((K,), jnp.int32),
        pltpu.VMEM((K,), jnp.int32),
        pltpu.VMEM((K,), jnp.int32),
        pltpu.VMEM((K,), jnp.int32),
        pltpu.VMEM((K,), jnp.int32),
        pltpu.VMEM((K,), jnp.int32),
        pltpu.VMEM((K,), jnp.int32),
        pltpu.VMEM((K,), jnp.int32),
        pltpu.VMEM((K * 16,), _f32),
        pltpu.VMEM((K, 256), _f32),
        pltpu.VMEM((K, 256), _f32),
        pltpu.VMEM((K, AW), _f32),
        pltpu.VMEM((K, AW), _f32),
        pltpu.VMEM((K, 128), _f32),
        pltpu.VMEM((K, 128), _f32),
        pltpu.VMEM((K, 128), _f32),
        pltpu.VMEM_SHARED((ACCR, 128), _f32),
        pltpu.SemaphoreType.DMA,
        pltpu.SemaphoreType.DMA,
        pltpu.SemaphoreType.DMA,
        pltpu.SemaphoreType.DMA,
        pltpu.SemaphoreType.DMA,
        pltpu.SemaphoreType.DMA,
        pltpu.SemaphoreType.DMA,
    ],
)(_edge_body)


def _att_mat(a):
    m = jnp.zeros((D, AW), _f32)
    for hh in range(H):
        m = m.at[hh * OUT:(hh + 1) * OUT, hh].set(a[hh])
    return m


def kernel(feature, edge_index, W1, att_src1, att_dst1, b1, g1, bt1,
           W2, att_src2, att_dst2, b2, g2, bt2):
    x = feature.reshape(B * N, D)
    x = jnp.concatenate([x[:N], jnp.zeros((NP - N, D), _f32),
                         x[N:], jnp.zeros((NP - N, D), _f32)], axis=0)
    ei = edge_index.astype(jnp.int32)
    padz = jnp.zeros((EP - E,), jnp.int32)
    padd = jnp.full((EP - E,), N + 1, jnp.int32)
    src2 = jnp.concatenate([ei[0], padz, ei[0] + jnp.int32(NP), padz])
    dst = jnp.concatenate([ei[1], padd])

    rden = jnp.concatenate(
        [jnp.kron(jnp.eye(H, dtype=_f32), jnp.ones((1, OUT), _f32)),
         jnp.zeros((DW - H, D), _f32)], axis=0)

    hx1, adp1 = _proj(x, W1, _att_mat(att_src1), _att_mat(att_dst1))
    msg1, dpack1 = _edge_call(hx1, adp1, src2, dst)
    hx2, adp2 = _mid(msg1, dpack1.reshape(NTP, DW), rden,
                     b1.reshape(1, D), g1.reshape(1, D),
                     bt1.reshape(1, D), W2, _att_mat(att_src2),
                     _att_mat(att_dst2))
    msg2, dpack2 = _edge_call(hx2, adp2, src2, dst)
    out = _last(msg2, dpack2.reshape(NTP, DW), rden, b2.reshape(1, D),
                g2.reshape(1, D), bt2.reshape(1, D))
    return out.reshape(B, NP, H * OUT)[:, :N, :]
